# baseline (speedup 1.0000x reference)
import functools

import jax
import jax.numpy as jnp
from jax import lax
from jax.experimental import pallas as pl
from jax.experimental.pallas import tpu as pltpu, tpu_sc as plsc

LANES = 16
SPLIT = 1
NBUF = 4


def _tree_sum(vals):
    while len(vals) > 1:
        nxt = [vals[i] + vals[i + 1] for i in range(0, len(vals) - 1, 2)]
        if len(vals) % 2:
            nxt.append(vals[-1])
        vals = nxt
    return vals[0]


@functools.lru_cache(maxsize=None)
def _make_kernel(n_res: int, t: int, d: int):
    info = plsc.get_sparse_core_info()
    nw = info.num_cores * info.num_subcores
    res_per_w = n_res // nw
    seg = t // SPLIT
    chunks_per_w = res_per_w * SPLIT
    assert t % SPLIT == 0 and NBUF % SPLIT == 0
    assert chunks_per_w % NBUF == 0

    mesh = plsc.VectorSubcoreMesh(core_axis_name="c", subcore_axis_name="s")

    @functools.partial(
        pl.kernel,
        mesh=mesh,
        out_type=jax.ShapeDtypeStruct((n_res, d), jnp.float32),
        scratch_types=[
            pltpu.VMEM((chunks_per_w * 2, seg // 2), jnp.int32),
            pltpu.VMEM((NBUF * 2, seg // 2, d), jnp.float32),
            pltpu.VMEM((res_per_w, d), jnp.float32),
        ] + [pltpu.SemaphoreType.DMA] * (2 * NBUF),
    )
    def k(idx_hbm, table_hbm, out_hbm, idx_v, rows_v, out_v, *sems):
        wid = lax.axis_index("s") * info.num_cores + lax.axis_index("c")
        pltpu.sync_copy(
            idx_hbm.at[pl.ds(wid * chunks_per_w * 2, chunks_per_w * 2)],
            idx_v)

        half = seg // 2

        def dmas(j, b):
            return [
                pltpu.make_async_copy(
                    table_hbm.at[idx_v.at[2 * j + h]],
                    rows_v.at[2 * b + h],
                    sems[2 * b + h])
                for h in range(2)
            ]

        def start(j, b):
            for c in dmas(j, b):
                c.start()

        def wait(j, b):
            for c in dmas(j, b):
                c.wait()

        for b in range(NBUF - 1):
            start(b, b)

        def body(g, _):
            part = [None] * (d // LANES)
            for b in range(NBUF):
                j = g * NBUF + b
                nxt = j + NBUF - 1

                @pl.when(nxt < chunks_per_w)
                def _():
                    start(nxt, (b + NBUF - 1) % NBUF)

                wait(j, b)
                for dc in range(d // LANES):
                    sl = pl.ds(dc * LANES, LANES)
                    s = _tree_sum([rows_v[2 * b + tt // half, tt % half, sl]
                                   for tt in range(seg)])
                    if b % SPLIT != SPLIT - 1:
                        part[dc] = s if b % SPLIT == 0 else part[dc] + s
                    else:
                        if SPLIT > 1:
                            s = part[dc] + s
                        out_v[g * (NBUF // SPLIT) + b // SPLIT, sl] = s
            return 0

        lax.fori_loop(0, chunks_per_w // NBUF, body, 0)
        pltpu.sync_copy(out_v, out_hbm.at[pl.ds(wid * res_per_w, res_per_w)])

    return k


def kernel(go_terms, table):
    b, l, t = go_terms.shape
    d = table.shape[1]
    n_res = b * l
    idx = go_terms.reshape(n_res * SPLIT * 2, t // (SPLIT * 2))
    out = _make_kernel(n_res, t, d)(idx, table)
    return out.reshape(b, l, d)


# speedup vs baseline: 1.0142x; 1.0142x over previous
import functools

import jax
import jax.numpy as jnp
from jax import lax
from jax.experimental import pallas as pl
from jax.experimental.pallas import tpu as pltpu, tpu_sc as plsc

LANES = 16
NBUF = 4
WB_BLOCKS = 4


def _tree_sum(vals):
    while len(vals) > 1:
        nxt = [vals[i] + vals[i + 1] for i in range(0, len(vals) - 1, 2)]
        if len(vals) % 2:
            nxt.append(vals[-1])
        vals = nxt
    return vals[0]


@functools.lru_cache(maxsize=None)
def _make_kernel(n_res: int, t: int, d: int):
    info = plsc.get_sparse_core_info()
    nw = info.num_cores * info.num_subcores
    res_per_w = n_res // nw
    wb = res_per_w // WB_BLOCKS
    gs_per_wb = wb // NBUF
    assert res_per_w % (NBUF * WB_BLOCKS) == 0

    mesh = plsc.VectorSubcoreMesh(core_axis_name="c", subcore_axis_name="s")

    @functools.partial(
        pl.kernel,
        mesh=mesh,
        out_type=jax.ShapeDtypeStruct((n_res, d), jnp.float32),
        scratch_types=[
            pltpu.VMEM((res_per_w, t), jnp.int32),
            pltpu.VMEM((NBUF, t, d), jnp.float32),
            pltpu.VMEM((res_per_w, d), jnp.float32),
            pltpu.SemaphoreType.DMA,
        ] + [pltpu.SemaphoreType.DMA] * NBUF,
    )
    def k(idx_hbm, table_hbm, out_hbm, idx_v, rows_v, out_v, wsem, *sems):
        wid = lax.axis_index("s") * info.num_cores + lax.axis_index("c")
        pltpu.sync_copy(idx_hbm.at[pl.ds(wid * res_per_w, res_per_w)], idx_v)

        def dma(j, b):
            return pltpu.make_async_copy(
                table_hbm.at[idx_v.at[j]], rows_v.at[b], sems[b])

        def wb_dma(m):
            return pltpu.make_async_copy(
                out_v.at[pl.ds(m * wb, wb)],
                out_hbm.at[pl.ds(wid * res_per_w + m * wb, wb)], wsem)

        for b in range(NBUF - 1):
            dma(b, b).start()

        def body(g, _):
            for b in range(NBUF):
                j = g * NBUF + b
                nxt = j + NBUF - 1

                @pl.when(nxt < res_per_w)
                def _():
                    dma(nxt, (b + NBUF - 1) % NBUF).start()

                dma(j, b).wait()
                for dc in range(d // LANES):
                    sl = pl.ds(dc * LANES, LANES)
                    out_v[j, sl] = _tree_sum(
                        [rows_v[b, tt, sl] for tt in range(t)])

            @pl.when(g % gs_per_wb == gs_per_wb - 1)
            def _():
                wb_dma(g // gs_per_wb).start()

            return 0

        lax.fori_loop(0, res_per_w // NBUF, body, 0)
        for m in range(WB_BLOCKS):
            wb_dma(m).wait()

    return k


def kernel(go_terms, table):
    b, l, t = go_terms.shape
    d = table.shape[1]
    n_res = b * l
    idx = go_terms.reshape(n_res, t)
    out = _make_kernel(n_res, t, d)(idx, table)
    return out.reshape(b, l, d)
